# baseline (speedup 1.0000x reference)
import jax
import jax.numpy as jnp
from jax import lax
from jax.experimental import pallas as pl
from jax.experimental.pallas import tpu as pltpu

_SLOPE = 0.01
_CDT = jnp.bfloat16


def _act(acc, s_ref, t_ref):
    y = acc * s_ref[...] + t_ref[...]
    return jnp.where(y > 0, y, _SLOPE * y).astype(_CDT)


def _col_masks(hw, w_img):
    p = lax.broadcasted_iota(jnp.int32, (1, hw), 1)
    col = jnp.bitwise_and(p, w_img - 1) if (w_img & (w_img - 1)) == 0 \
        else p % w_img
    return col == 0, col == (w_img - 1)


def _store_x(scr_ref, x, hw):
    c = x.shape[0]
    scr_ref[:c, :hw] = x
    scr_ref[:c, hw:2 * hw] = x


def _tap_stack(scr_ref, c, hw, w_img):
    is_first, is_last = _col_masks(hw, w_img)
    taps = []
    for ky in range(3):
        for kx in range(3):
            dy, dx = ky - 1, kx - 1
            s = (dy * w_img + dx) % hw
            tap = scr_ref[:c, s:s + hw]
            if dx == 1:
                sf = (s - w_img) % hw
                tap = jnp.where(is_last, scr_ref[:c, sf:sf + hw], tap)
            elif dx == -1:
                sf = (s + w_img) % hw
                tap = jnp.where(is_first, scr_ref[:c, sf:sf + hw], tap)
            taps.append(tap)
    return jnp.concatenate(taps, axis=0)


def _kdot(w_ref, cols):
    return lax.dot_general(w_ref[...], cols, (((1,), (0,)), ((), ())),
                           preferred_element_type=jnp.float32)


def _conv3x3(scr_ref, x, w_ref, w_img):
    c, hw = x.shape
    _store_x(scr_ref, x, hw)
    return _kdot(w_ref, _tap_stack(scr_ref, c, hw, w_img))


def _block(x, scr_ref, pr, w_img):
    w1, s1, t1, w2, s2, t2, w3, s3, t3 = pr
    hw = x.shape[1]
    h = _act(_conv3x3(scr_ref, x, w1, w_img), s1, t1)
    h = _act(_conv3x3(scr_ref, h, w2, w_img), s2, t2)
    return _act(_kdot(w3, h), s3, t3)


def _merged_block(scr_ref, c_in, hw, pr, w_img):
    w1, s1, t1, w2, s2, t2, w3, s3, t3 = pr
    h = _act(_kdot(w1, _tap_stack(scr_ref, c_in, hw, w_img)), s1, t1)
    h = _act(_conv3x3(scr_ref, h, w2, w_img), s2, t2)
    return _act(_kdot(w3, h), s3, t3)


def _upsample_into_scr(scr_ref, y, h_img, w_img):
    c = y.shape[0]
    w2 = 2 * w_img
    yw = jnp.repeat(y, 2, axis=1)
    for i in range(h_img):
        row = yw[:, i * w2:(i + 1) * w2]
        scr_ref[:c, (2 * i) * w2:(2 * i + 1) * w2] = row
        scr_ref[:c, (2 * i + 1) * w2:(2 * i + 2) * w2] = row
    hw4 = 4 * h_img * w_img
    scr_ref[:c, hw4:2 * hw4] = scr_ref[:c, :hw4]


def _pyramid_kernel(*refs):
    zs = refs[:6]
    out_ref, scr_ref = refs[-2], refs[-1]
    pr = list(refs[6:-2])
    pos = [0]

    def take(n):
        v = pr[pos[0]:pos[0] + n]
        pos[0] += n
        return v

    cb1 = take(9)
    level_ps = [(take(9), take(9)) for _ in range(5)]
    lw, lb = take(2)

    h_img = w_img = 4
    y = _block(zs[0][0], scr_ref, cb1, w_img)
    for lvl in range(5):
        skip_p, main_p = level_ps[lvl]
        skip = _block(zs[1 + lvl][0], scr_ref, skip_p, 2 * w_img)
        _upsample_into_scr(scr_ref, y, h_img, w_img)
        h_img, w_img = 2 * h_img, 2 * w_img
        hw = h_img * w_img
        c1, c2 = y.shape[0], skip.shape[0]
        scr_ref[c1:c1 + c2, :hw] = skip
        scr_ref[c1:c1 + c2, hw:2 * hw] = skip
        y = _merged_block(scr_ref, c1 + c2, hw, main_p, w_img)
    out = _kdot(lw, y) + lb[...]
    out_ref[0] = out.astype(out_ref.dtype)


def _pack3(w):
    return jnp.transpose(w, (1, 0, 2)).reshape(w.shape[1], -1).astype(_CDT)


def _prep_block(w1, s1, t1, w2, s2, t2, w3, s3, t3):
    return [_pack3(w1), s1, t1, _pack3(w2), s2, t2, w3.astype(_CDT), s3, t3]


def _act_spec(c, hw):
    return pl.BlockSpec((1, c, hw), lambda n: (n, 0, 0))


def _param_spec(arr):
    zeros = (0,) * arr.ndim
    return pl.BlockSpec(arr.shape, lambda n, _z=zeros: _z)


def kernel(cb1_1__w1, cb1_1__s1, cb1_1__t1, cb1_1__w2, cb1_1__s2, cb1_1__t2, cb1_1__w3, cb1_1__s3, cb1_1__t3, cb2_1__w1, cb2_1__s1, cb2_1__t1, cb2_1__w2, cb2_1__s2, cb2_1__t2, cb2_1__w3, cb2_1__s3, cb2_1__t3, cb2_2__w1a, cb2_2__w1b, cb2_2__s1, cb2_2__t1, cb2_2__w2, cb2_2__s2, cb2_2__t2, cb2_2__w3, cb2_2__s3, cb2_2__t3, cb3_1__w1, cb3_1__s1, cb3_1__t1, cb3_1__w2, cb3_1__s2, cb3_1__t2, cb3_1__w3, cb3_1__s3, cb3_1__t3, cb3_2__w1a, cb3_2__w1b, cb3_2__s1, cb3_2__t1, cb3_2__w2, cb3_2__s2, cb3_2__t2, cb3_2__w3, cb3_2__s3, cb3_2__t3, cb4_1__w1, cb4_1__s1, cb4_1__t1, cb4_1__w2, cb4_1__s2, cb4_1__t2, cb4_1__w3, cb4_1__s3, cb4_1__t3, cb4_2__w1a, cb4_2__w1b, cb4_2__s1, cb4_2__t1, cb4_2__w2, cb4_2__s2, cb4_2__t2, cb4_2__w3, cb4_2__s3, cb4_2__t3, cb5_1__w1, cb5_1__s1, cb5_1__t1, cb5_1__w2, cb5_1__s2, cb5_1__t2, cb5_1__w3, cb5_1__s3, cb5_1__t3, cb5_2__w1a, cb5_2__w1b, cb5_2__s1, cb5_2__t1, cb5_2__w2, cb5_2__s2, cb5_2__t2, cb5_2__w3, cb5_2__s3, cb5_2__t3, cb6_1__w1, cb6_1__s1, cb6_1__t1, cb6_1__w2, cb6_1__s2, cb6_1__t2, cb6_1__w3, cb6_1__s3, cb6_1__t3, cb6_2__w1a, cb6_2__w1b, cb6_2__s1, cb6_2__t1, cb6_2__w2, cb6_2__s2, cb6_2__t2, cb6_2__w3, cb6_2__s3, cb6_2__t3, last__w, last__b, z0, z1, z2, z3, z4, z5):
    n = z0.shape[0]

    zs = []
    for zi in (z5, z4, z3, z2, z1, z0):
        zs.append(zi.reshape(zi.shape[0], zi.shape[1], -1).astype(_CDT))

    params = []
    params += _prep_block(cb1_1__w1, cb1_1__s1, cb1_1__t1, cb1_1__w2,
                          cb1_1__s2, cb1_1__t2, cb1_1__w3, cb1_1__s3, cb1_1__t3)
    merged = [
        (cb2_1__w1, cb2_1__s1, cb2_1__t1, cb2_1__w2, cb2_1__s2, cb2_1__t2, cb2_1__w3, cb2_1__s3, cb2_1__t3,
         cb2_2__w1a, cb2_2__w1b, cb2_2__s1, cb2_2__t1, cb2_2__w2, cb2_2__s2, cb2_2__t2, cb2_2__w3, cb2_2__s3, cb2_2__t3),
        (cb3_1__w1, cb3_1__s1, cb3_1__t1, cb3_1__w2, cb3_1__s2, cb3_1__t2, cb3_1__w3, cb3_1__s3, cb3_1__t3,
         cb3_2__w1a, cb3_2__w1b, cb3_2__s1, cb3_2__t1, cb3_2__w2, cb3_2__s2, cb3_2__t2, cb3_2__w3, cb3_2__s3, cb3_2__t3),
        (cb4_1__w1, cb4_1__s1, cb4_1__t1, cb4_1__w2, cb4_1__s2, cb4_1__t2, cb4_1__w3, cb4_1__s3, cb4_1__t3,
         cb4_2__w1a, cb4_2__w1b, cb4_2__s1, cb4_2__t1, cb4_2__w2, cb4_2__s2, cb4_2__t2, cb4_2__w3, cb4_2__s3, cb4_2__t3),
        (cb5_1__w1, cb5_1__s1, cb5_1__t1, cb5_1__w2, cb5_1__s2, cb5_1__t2, cb5_1__w3, cb5_1__s3, cb5_1__t3,
         cb5_2__w1a, cb5_2__w1b, cb5_2__s1, cb5_2__t1, cb5_2__w2, cb5_2__s2, cb5_2__t2, cb5_2__w3, cb5_2__s3, cb5_2__t3),
        (cb6_1__w1, cb6_1__s1, cb6_1__t1, cb6_1__w2, cb6_1__s2, cb6_1__t2, cb6_1__w3, cb6_1__s3, cb6_1__t3,
         cb6_2__w1a, cb6_2__w1b, cb6_2__s1, cb6_2__t1, cb6_2__w2, cb6_2__s2, cb6_2__t2, cb6_2__w3, cb6_2__s3, cb6_2__t3),
    ]
    for (sw1, ss1, st1, sw2, ss2, st2, sw3, ss3, st3,
         mw1a, mw1b, ms1, mt1, mw2, ms2, mt2, mw3, ms3, mt3) in merged:
        params += _prep_block(sw1, ss1, st1, sw2, ss2, st2, sw3, ss3, st3)
        mw1 = jnp.concatenate([mw1a, mw1b], axis=2)
        params += _prep_block(mw1, ms1, mt1, mw2, ms2, mt2, mw3, ms3, mt3)
    params += [last__w.astype(_CDT), last__b]

    hw_top = zs[-1].shape[2]
    in_specs = ([_act_spec(3, z.shape[2]) for z in zs]
                + [_param_spec(a) for a in params])
    out = pl.pallas_call(
        _pyramid_kernel,
        out_shape=jax.ShapeDtypeStruct((n, 3, hw_top), jnp.float32),
        grid=(n,),
        in_specs=in_specs,
        out_specs=_act_spec(3, hw_top),
        scratch_shapes=[pltpu.VMEM((48, 2 * hw_top), _CDT)],
        compiler_params=pltpu.CompilerParams(
            dimension_semantics=("parallel",),
            vmem_limit_bytes=56 * 1024 * 1024,
        ),
    )(*zs, *params)
    s = z0.shape[2]
    return out.reshape(n, 3, s, s)
